# baseline (speedup 1.0000x reference)
import functools

import jax
import jax.numpy as jnp
from jax import lax
from jax.experimental import pallas as pl
from jax.experimental.pallas import tpu as pltpu
from jax.experimental.pallas import tpu_sc as plsc

NC = 2
NS = 16
L = 16
NW = NC * NS

B, S, D = 4, 4096, 1024
S_PER_W = S // NW
CH = 16
N_CH = S_PER_W // CH
CHW = CH * D
VEC_ITERS = CHW // L

_mesh = plsc.VectorSubcoreMesh(core_axis_name="c", subcore_axis_name="s")


@functools.partial(
    pl.kernel,
    out_type=jax.ShapeDtypeStruct((B * S * D,), jnp.float32),
    mesh=_mesh,
    scratch_types=[
        pltpu.VMEM((CHW,), jnp.float32),
        pltpu.VMEM((CHW,), jnp.float32),
    ],
)
def _pos_add(x_hbm, pos_hbm, out_hbm, pos_v, x_v):
    wid = lax.axis_index("s") * NC + lax.axis_index("c")
    pbase = wid * (S_PER_W * D)
    for c in range(N_CH):
        off = pbase + c * CHW
        pltpu.sync_copy(pos_hbm.at[pl.ds(off, CHW)], pos_v)
        for b in range(B):
            xoff = b * (S * D) + off
            pltpu.sync_copy(x_hbm.at[pl.ds(xoff, CHW)], x_v)

            @pl.loop(0, VEC_ITERS, unroll=8)
            def _add(i):
                sl = pl.ds(i * L, L)
                x_v[sl] = x_v[sl] + pos_v[sl]

            pltpu.sync_copy(x_v, out_hbm.at[pl.ds(xoff, CHW)])


def kernel(x, pos_table):
    seq_len = x.shape[1]
    flat = _pos_add(
        x.reshape(-1),
        pos_table[:seq_len].reshape(-1),
    )
    return flat.reshape(x.shape)
